# baseline (speedup 1.0000x reference)
import functools

import jax
import jax.numpy as jnp
from jax import lax
from jax.experimental import pallas as pl
from jax.experimental.pallas import tpu as pltpu
from jax.experimental.pallas import tpu_sc as plsc

B = 2048
HID = 1024
E = 8
GD = 16
COEF = 0.01
M = 256
NPAD = 4096 + E * M
NB = NPAD // M
NC = 2
NS = 16
PT = 4096 // NS
SLT = NPAD // NS
GR = NPAD // (NC * NS)
GC = 48


def _gate_body(x_ref, Wg1_ref, bg1_ref, Wg2_ref, bg2_ref, Wsh_ref, bsh_ref,
               shw_ref, rtw_ref, noise_ref,
               base_ref, tke_ref, tkw_ref, loss_ref):
    x = x_ref[...]
    h = jnp.maximum(
        jnp.dot(x, Wg1_ref[...].T, preferred_element_type=jnp.float32)
        + bg1_ref[...], 0.0)
    logits = (jnp.dot(h, Wg2_ref[...].T, preferred_element_type=jnp.float32)
              + bg2_ref[...])
    gw = jax.nn.softmax(logits, axis=-1) + noise_ref[...]
    e_iota = jax.lax.broadcasted_iota(jnp.int32, (B, E), 1)
    i1 = jnp.argmax(gw, axis=1)
    m1 = jnp.max(gw, axis=1)
    masked = jnp.where(e_iota == i1[:, None], -jnp.inf, gw)
    i2 = jnp.argmax(masked, axis=1)
    m2 = jnp.max(masked, axis=1)
    denom = m1 + m2
    rtw = rtw_ref[0, 0]
    tke_ref[...] = jnp.concatenate(
        [i1.astype(jnp.int32).reshape(1, B), i2.astype(jnp.int32).reshape(1, B)], axis=0)
    tkw_ref[...] = jnp.concatenate(
        [(rtw * m1 / denom).reshape(1, B), (rtw * m2 / denom).reshape(1, B)],
        axis=0)
    sh = jax.nn.sigmoid(
        jnp.dot(x, Wsh_ref[...].T, preferred_element_type=jnp.float32)
        + bsh_ref[...])
    base_ref[...] = (1.0 + rtw) * x + shw_ref[0, 0] * sh
    p = jnp.sum(gw, axis=0, keepdims=True) * (1.0 / B)
    loss_ref[...] = jnp.reshape(jnp.mean((1.0 / E - p) ** 2) * COEF, (1, 1))


def _sort_gather_body(tke, tkw, x,
                      xg, dests, wvec, bexp,
                      evb, wvb, dvb, gvb, cntv, allc, zeroi, zerof,
                      ibuf, rbuf, bbuf,
                      cnts_sh, gidx_sh, wv_sh, sem):
    c = lax.axis_index("c")
    si = lax.axis_index("s")
    row = si // 8
    col = (si % 8) * PT
    iota = jax.lax.iota(jnp.int32, 16)
    zi = jnp.zeros((16,), jnp.int32)

    for q in range(2):
        pltpu.sync_copy(tke.at[row, pl.ds(col + q * 128, 128)], evb.at[q])
        pltpu.sync_copy(tkw.at[row, pl.ds(col + q * 128, 128)], wvb.at[q])
    ev = [evb[q, pl.ds(r * 16, 16)] for q in range(2) for r in range(8)]

    cnts = zi
    for e in range(E):
        tot_e = jnp.int32(0)
        for v in ev:
            tot_e = tot_e + plsc.cumsum(jnp.where(v == e, 1, 0))[15]
        cnts = jnp.where(iota == e, jnp.full((16,), tot_e, jnp.int32), cnts)
    cntv[...] = cnts
    pltpu.sync_copy(cntv, cnts_sh.at[si])
    plsc.subcore_barrier()

    pltpu.sync_copy(cnts_sh, allc)
    tot = zi
    pw = zi
    sivec = jnp.full((16,), si, jnp.int32)
    for w in range(NS):
        rw = allc[w]
        tot = tot + rw
        pw = pw + jnp.where(jnp.full((16,), w, jnp.int32) < sivec, rw, zi)
    pc = ((tot + (M - 1)) >> 8) << 8
    cum = plsc.cumsum(pc)
    padoff = cum - pc
    basev = padoff + pw

    dv = [zi] * 16
    for e in range(E):
        run = basev[e]
        for j, v in enumerate(ev):
            mk = v == e
            cs = plsc.cumsum(jnp.where(mk, 1, 0))
            dv[j] = jnp.where(mk, run + cs - 1, dv[j])
            run = run + cs[15]
    for q in range(2):
        for r in range(8):
            dvb[q, pl.ds(r * 16, 16)] = dv[q * 8 + r]

    @pl.when(c == 0)
    def _():
        for q in range(2):
            pltpu.sync_copy(dvb.at[q], dests.at[row, pl.ds(col + q * 128, 128)])

    for r in range(SLT // 16):
        zeroi[pl.ds(r * 16, 16)] = zi
        zerof[pl.ds(r * 16, 16)] = jnp.zeros((16,), jnp.float32)
    pltpu.sync_copy(zeroi, gidx_sh.at[pl.ds(si * SLT, SLT)])
    pltpu.sync_copy(zerof, wv_sh.at[pl.ds(si * SLT, SLT)])
    plsc.subcore_barrier()

    for q in range(2):
        for r in range(8):
            gvb[q, pl.ds(r * 16, 16)] = col + (q * 128 + r * 16) + iota
    for q in range(2):
        pltpu.sync_copy(gvb.at[q], gidx_sh.at[dvb.at[q]])
        pltpu.sync_copy(wvb.at[q], wv_sh.at[dvb.at[q]])
    plsc.subcore_barrier()

    @pl.when(c == 0)
    def _():
        pltpu.sync_copy(wv_sh.at[pl.ds(si * SLT, SLT)],
                        wvec.at[pl.ds(si * SLT, SLT)])

    @pl.when((c == 0) & (si == 0))
    def _():
        for v in range(2):
            bidx = (jnp.full((16,), v * 16, jnp.int32) + iota) * M
            acc = zi
            for e in range(E):
                acc = acc + jnp.where(bidx >= cum[e], 1, 0)
            bbuf[pl.ds(v * 16, 16)] = jnp.minimum(acc, E - 1)
        pltpu.sync_copy(bbuf, bexp)

    wid = c * NS + si
    r0 = wid * GR
    pltpu.sync_copy(gidx_sh.at[pl.ds(r0, GR)], ibuf)
    for k in range(GR // GC):
        pltpu.async_copy(x.at[ibuf.at[pl.ds(k * GC, GC)]], rbuf, sem).wait()
        pltpu.sync_copy(rbuf, xg.at[pl.ds(r0 + k * GC, GC)])


def _expert_body(bexp_ref, xg_ref, We_ref, be_ref, wv_ref, ysel_ref):
    z = (jnp.dot(xg_ref[...], We_ref[0].T, preferred_element_type=jnp.float32)
         + be_ref[0])
    ysel_ref[...] = wv_ref[...] * jax.nn.sigmoid(z)


def _combine_body(ysel, dests, basev, out,
                  i1b, i2b, r1, r2, bb, ob, sem):
    c = lax.axis_index("c")
    si = lax.axis_index("s")
    wid = c * NS + si
    t0 = wid * (B // (NC * NS))
    for k in range(4):
        tk0 = t0 + k * 16
        pltpu.sync_copy(dests.at[0, pl.ds(tk0, 16)], i1b)
        pltpu.sync_copy(dests.at[1, pl.ds(tk0, 16)], i2b)
        pltpu.async_copy(ysel.at[i1b], r1, sem).wait()
        pltpu.async_copy(ysel.at[i2b], r2, sem).wait()
        pltpu.sync_copy(basev.at[pl.ds(tk0, 16)], bb)

        def _row(i, carry):
            for j in range(HID // 16):
                sl = pl.ds(j * 16, 16)
                ob[i, sl] = bb[i, sl] + r1[i, sl] + r2[i, sl]
            return carry

        lax.fori_loop(0, 16, _row, 0)
        pltpu.sync_copy(ob, out.at[pl.ds(tk0, 16)])


def kernel(x, W_g1, b_g1, W_g2, b_g2, W_sh, b_sh, sh_w, rt_w, W_e, b_e, noise):
    const2 = lambda *_: (0, 0)
    base, tke, tkw, loss = pl.pallas_call(
        _gate_body,
        grid=(1,),
        in_specs=[
            pl.BlockSpec((B, HID), const2),
            pl.BlockSpec((GD, HID), const2),
            pl.BlockSpec((1, GD), const2),
            pl.BlockSpec((E, GD), const2),
            pl.BlockSpec((1, E), const2),
            pl.BlockSpec((HID, HID), const2),
            pl.BlockSpec((1, HID), const2),
            pl.BlockSpec((1, 1), const2),
            pl.BlockSpec((1, 1), const2),
            pl.BlockSpec((B, E), const2),
        ],
        out_specs=[
            pl.BlockSpec((B, HID), const2),
            pl.BlockSpec((2, B), const2),
            pl.BlockSpec((2, B), const2),
            pl.BlockSpec((1, 1), const2),
        ],
        out_shape=[
            jax.ShapeDtypeStruct((B, HID), jnp.float32),
            jax.ShapeDtypeStruct((2, B), jnp.int32),
            jax.ShapeDtypeStruct((2, B), jnp.float32),
            jax.ShapeDtypeStruct((1, 1), jnp.float32),
        ],
    )(x, W_g1, b_g1.reshape(1, GD), W_g2, b_g2.reshape(1, E),
      W_sh, b_sh.reshape(1, HID), sh_w.reshape(1, 1), rt_w.reshape(1, 1),
      noise)

    mesh = plsc.VectorSubcoreMesh(core_axis_name="c", subcore_axis_name="s",
                                  num_cores=NC, num_subcores=NS)
    xg, dests, wvec, bexp = pl.kernel(
        _sort_gather_body,
        out_type=[
            jax.ShapeDtypeStruct((NPAD, HID), jnp.float32),
            jax.ShapeDtypeStruct((2, B), jnp.int32),
            jax.ShapeDtypeStruct((NPAD,), jnp.float32),
            jax.ShapeDtypeStruct((32,), jnp.int32),
        ],
        mesh=mesh,
        compiler_params=pltpu.CompilerParams(needs_layout_passes=False),
        scratch_types=[
            pltpu.VMEM((2, 128), jnp.int32),
            pltpu.VMEM((2, 128), jnp.float32),
            pltpu.VMEM((2, 128), jnp.int32),
            pltpu.VMEM((2, 128), jnp.int32),
            pltpu.VMEM((16,), jnp.int32),
            pltpu.VMEM((NS, 16), jnp.int32),
            pltpu.VMEM((SLT,), jnp.int32),
            pltpu.VMEM((SLT,), jnp.float32),
            pltpu.VMEM((GR,), jnp.int32),
            pltpu.VMEM((GC, HID), jnp.float32),
            pltpu.VMEM((32,), jnp.int32),
            pltpu.VMEM_SHARED((NS, 16), jnp.int32),
            pltpu.VMEM_SHARED((NPAD,), jnp.int32),
            pltpu.VMEM_SHARED((NPAD,), jnp.float32),
            pltpu.SemaphoreType.DMA,
        ],
    )(tke, tkw, x)

    ysel = pl.pallas_call(
        _expert_body,
        grid_spec=pltpu.PrefetchScalarGridSpec(
            num_scalar_prefetch=1,
            grid=(NB,),
            in_specs=[
                pl.BlockSpec((M, HID), lambda i, be: (i, 0)),
                pl.BlockSpec((1, HID, HID), lambda i, be: (be[i], 0, 0)),
                pl.BlockSpec((1, 1, HID), lambda i, be: (be[i], 0, 0)),
                pl.BlockSpec((M, 1), lambda i, be: (i, 0)),
            ],
            out_specs=pl.BlockSpec((M, HID), lambda i, be: (i, 0)),
        ),
        out_shape=jax.ShapeDtypeStruct((NPAD, HID), jnp.float32),
    )(bexp, xg, W_e, b_e.reshape(E, 1, HID), wvec.reshape(NPAD, 1))

    out = pl.kernel(
        _combine_body,
        out_type=jax.ShapeDtypeStruct((B, HID), jnp.float32),
        mesh=plsc.VectorSubcoreMesh(core_axis_name="c", subcore_axis_name="s",
                                    num_cores=NC, num_subcores=NS),
        scratch_types=[
            pltpu.VMEM((16,), jnp.int32),
            pltpu.VMEM((16,), jnp.int32),
            pltpu.VMEM((16, HID), jnp.float32),
            pltpu.VMEM((16, HID), jnp.float32),
            pltpu.VMEM((16, HID), jnp.float32),
            pltpu.VMEM((16, HID), jnp.float32),
            pltpu.SemaphoreType.DMA,
        ],
    )(ysel, dests, base)
    return out, loss[0, 0]


# speedup vs baseline: 1.0350x; 1.0350x over previous
import functools

import jax
import jax.numpy as jnp
from jax import lax
from jax.experimental import pallas as pl
from jax.experimental.pallas import tpu as pltpu
from jax.experimental.pallas import tpu_sc as plsc

B = 2048
HID = 1024
E = 8
GD = 16
COEF = 0.01
M = 256
NPAD = 4096 + E * M
NB = NPAD // M
NC = 2
NS = 16
PT = 4096 // NS
SLT = NPAD // NS
GR = NPAD // (NC * NS)
GC = 48


def _gate_body(x_ref, Wg1_ref, bg1_ref, Wg2_ref, bg2_ref, rtw_ref, noise_ref,
               tke_ref, tkw_ref, loss_ref):
    x = x_ref[...]
    h = jnp.maximum(
        jnp.dot(x, Wg1_ref[...].T, preferred_element_type=jnp.float32)
        + bg1_ref[...], 0.0)
    logits = (jnp.dot(h, Wg2_ref[...].T, preferred_element_type=jnp.float32)
              + bg2_ref[...])
    gw = jax.nn.softmax(logits, axis=-1) + noise_ref[...]
    e_iota = jax.lax.broadcasted_iota(jnp.int32, (B, E), 1)
    i1 = jnp.argmax(gw, axis=1)
    m1 = jnp.max(gw, axis=1)
    masked = jnp.where(e_iota == i1[:, None], -jnp.inf, gw)
    i2 = jnp.argmax(masked, axis=1)
    m2 = jnp.max(masked, axis=1)
    denom = m1 + m2
    rtw = rtw_ref[0, 0]
    tke_ref[...] = jnp.concatenate(
        [i1.astype(jnp.int32).reshape(1, B), i2.astype(jnp.int32).reshape(1, B)], axis=0)
    tkw_ref[...] = jnp.concatenate(
        [(rtw * m1 / denom).reshape(1, B), (rtw * m2 / denom).reshape(1, B)],
        axis=0)
    p = jnp.sum(gw, axis=0, keepdims=True) * (1.0 / B)
    loss_ref[...] = jnp.reshape(jnp.mean((1.0 / E - p) ** 2) * COEF, (1, 1))


def _base_body(x_ref, Wsh_ref, bsh_ref, shw_ref, rtw_ref, base_ref):
    x = x_ref[...]
    sh = jax.nn.sigmoid(
        jnp.dot(x, Wsh_ref[...].T, preferred_element_type=jnp.float32)
        + bsh_ref[...])
    base_ref[...] = (1.0 + rtw_ref[0, 0]) * x + shw_ref[0, 0] * sh


def _sort_gather_body(tke, tkw, x,
                      xg, dests, wvec, bexp,
                      evb, wvb, dvb, gvb, cntv, allc, zeroi, zerof,
                      ibuf, rbuf, rbuf2, bbuf,
                      cnts_sh, gidx_sh, wv_sh, sem, semg2, semw, semw2):
    c = lax.axis_index("c")
    si = lax.axis_index("s")
    row = si // 8
    col = (si % 8) * PT
    iota = jax.lax.iota(jnp.int32, 16)
    zi = jnp.zeros((16,), jnp.int32)

    for q in range(2):
        pltpu.sync_copy(tke.at[row, pl.ds(col + q * 128, 128)], evb.at[q])
        pltpu.sync_copy(tkw.at[row, pl.ds(col + q * 128, 128)], wvb.at[q])
    ev = [evb[q, pl.ds(r * 16, 16)] for q in range(2) for r in range(8)]

    cnts = zi
    for e in range(E):
        tot_e = jnp.int32(0)
        for v in ev:
            tot_e = tot_e + plsc.cumsum(jnp.where(v == e, 1, 0))[15]
        cnts = jnp.where(iota == e, jnp.full((16,), tot_e, jnp.int32), cnts)
    cntv[...] = cnts
    pltpu.sync_copy(cntv, cnts_sh.at[si])
    plsc.subcore_barrier()

    pltpu.sync_copy(cnts_sh, allc)
    tot = zi
    pw = zi
    sivec = jnp.full((16,), si, jnp.int32)
    for w in range(NS):
        rw = allc[w]
        tot = tot + rw
        pw = pw + jnp.where(jnp.full((16,), w, jnp.int32) < sivec, rw, zi)
    pc = ((tot + (M - 1)) >> 8) << 8
    cum = plsc.cumsum(pc)
    padoff = cum - pc
    basev = padoff + pw

    dv = [zi] * 16
    for e in range(E):
        run = basev[e]
        for j, v in enumerate(ev):
            mk = v == e
            cs = plsc.cumsum(jnp.where(mk, 1, 0))
            dv[j] = jnp.where(mk, run + cs - 1, dv[j])
            run = run + cs[15]
    for q in range(2):
        for r in range(8):
            dvb[q, pl.ds(r * 16, 16)] = dv[q * 8 + r]

    @pl.when(c == 0)
    def _():
        for q in range(2):
            pltpu.sync_copy(dvb.at[q], dests.at[row, pl.ds(col + q * 128, 128)])

    for r in range(SLT // 16):
        zeroi[pl.ds(r * 16, 16)] = zi
        zerof[pl.ds(r * 16, 16)] = jnp.zeros((16,), jnp.float32)
    pltpu.sync_copy(zeroi, gidx_sh.at[pl.ds(si * SLT, SLT)])
    pltpu.sync_copy(zerof, wv_sh.at[pl.ds(si * SLT, SLT)])
    plsc.subcore_barrier()

    for q in range(2):
        for r in range(8):
            gvb[q, pl.ds(r * 16, 16)] = col + (q * 128 + r * 16) + iota
    for q in range(2):
        pltpu.sync_copy(gvb.at[q], gidx_sh.at[dvb.at[q]])
        pltpu.sync_copy(wvb.at[q], wv_sh.at[dvb.at[q]])
    plsc.subcore_barrier()

    @pl.when(c == 0)
    def _():
        pltpu.sync_copy(wv_sh.at[pl.ds(si * SLT, SLT)],
                        wvec.at[pl.ds(si * SLT, SLT)])

    @pl.when((c == 0) & (si == 0))
    def _():
        for v in range(2):
            bidx = (jnp.full((16,), v * 16, jnp.int32) + iota) * M
            acc = zi
            for e in range(E):
                acc = acc + jnp.where(bidx >= cum[e], 1, 0)
            bbuf[pl.ds(v * 16, 16)] = jnp.minimum(acc, E - 1)
        pltpu.sync_copy(bbuf, bexp)

    wid = c * NS + si
    r0 = wid * GR
    pltpu.sync_copy(gidx_sh.at[pl.ds(r0, GR)], ibuf)
    nk = GR // GC
    rbufs = (rbuf, rbuf2)
    gsems = (sem, semg2)
    wsems = (semw, semw2)
    gd = [None] * nk
    wd = [None] * nk
    for k in range(min(2, nk)):
        gd[k] = pltpu.async_copy(x.at[ibuf.at[pl.ds(k * GC, GC)]],
                                 rbufs[k % 2], gsems[k % 2])
    for k in range(nk):
        gd[k].wait()
        wd[k] = pltpu.async_copy(rbufs[k % 2], xg.at[pl.ds(r0 + k * GC, GC)],
                                 wsems[k % 2])
        if k + 2 < nk:
            wd[k].wait()
            gd[k + 2] = pltpu.async_copy(
                x.at[ibuf.at[pl.ds((k + 2) * GC, GC)]],
                rbufs[k % 2], gsems[k % 2])
    for k in range(max(nk - 2, 0), nk):
        wd[k].wait()


def _expert_body(bexp_ref, xg_ref, We_ref, be_ref, wv_ref, ysel_ref):
    z = (jnp.dot(xg_ref[...], We_ref[0].T, preferred_element_type=jnp.float32)
         + be_ref[0])
    ysel_ref[...] = wv_ref[...] * jax.nn.sigmoid(z)


def _combine_body(ysel, dests, basev, out,
                  i1b, i2b, r1a, r2a, r1b, r2b, bb, ob, sem, sem2):
    c = lax.axis_index("c")
    si = lax.axis_index("s")
    wid = c * NS + si
    t0 = wid * (B // (NC * NS))
    r1s = (r1a, r1b)
    r2s = (r2a, r2b)
    sems = (sem, sem2)
    nk = 4

    def fire(k):
        tk0 = t0 + k * 16
        pltpu.sync_copy(dests.at[0, pl.ds(tk0, 16)], i1b)
        pltpu.sync_copy(dests.at[1, pl.ds(tk0, 16)], i2b)
        g1 = pltpu.async_copy(ysel.at[i1b], r1s[k % 2], sems[k % 2])
        g2 = pltpu.async_copy(ysel.at[i2b], r2s[k % 2], sems[k % 2])
        return g1, g2

    pend = fire(0)
    for k in range(nk):
        tk0 = t0 + k * 16
        pend[0].wait()
        pend[1].wait()
        if k + 1 < nk:
            pend = fire(k + 1)
        pltpu.sync_copy(basev.at[pl.ds(tk0, 16)], bb)
        r1, r2 = r1s[k % 2], r2s[k % 2]

        def _row(i, carry):
            for j in range(HID // 16):
                sl = pl.ds(j * 16, 16)
                ob[i, sl] = bb[i, sl] + r1[i, sl] + r2[i, sl]
            return carry

        lax.fori_loop(0, 16, _row, 0)
        pltpu.sync_copy(ob, out.at[pl.ds(tk0, 16)])


def kernel(x, W_g1, b_g1, W_g2, b_g2, W_sh, b_sh, sh_w, rt_w, W_e, b_e, noise):
    const2 = lambda *_: (0, 0)
    tke, tkw, loss = pl.pallas_call(
        _gate_body,
        grid=(1,),
        in_specs=[
            pl.BlockSpec((B, HID), const2),
            pl.BlockSpec((GD, HID), const2),
            pl.BlockSpec((1, GD), const2),
            pl.BlockSpec((E, GD), const2),
            pl.BlockSpec((1, E), const2),
            pl.BlockSpec((1, 1), const2),
            pl.BlockSpec((B, E), const2),
        ],
        out_specs=[
            pl.BlockSpec((2, B), const2),
            pl.BlockSpec((2, B), const2),
            pl.BlockSpec((1, 1), const2),
        ],
        out_shape=[
            jax.ShapeDtypeStruct((2, B), jnp.int32),
            jax.ShapeDtypeStruct((2, B), jnp.float32),
            jax.ShapeDtypeStruct((1, 1), jnp.float32),
        ],
    )(x, W_g1, b_g1.reshape(1, GD), W_g2, b_g2.reshape(1, E),
      rt_w.reshape(1, 1), noise)

    base = pl.pallas_call(
        _base_body,
        grid=(1,),
        in_specs=[
            pl.BlockSpec((B, HID), const2),
            pl.BlockSpec((HID, HID), const2),
            pl.BlockSpec((1, HID), const2),
            pl.BlockSpec((1, 1), const2),
            pl.BlockSpec((1, 1), const2),
        ],
        out_specs=pl.BlockSpec((B, HID), const2),
        out_shape=jax.ShapeDtypeStruct((B, HID), jnp.float32),
    )(x, W_sh, b_sh.reshape(1, HID), sh_w.reshape(1, 1), rt_w.reshape(1, 1))

    mesh = plsc.VectorSubcoreMesh(core_axis_name="c", subcore_axis_name="s",
                                  num_cores=NC, num_subcores=NS)
    xg, dests, wvec, bexp = pl.kernel(
        _sort_gather_body,
        out_type=[
            jax.ShapeDtypeStruct((NPAD, HID), jnp.float32),
            jax.ShapeDtypeStruct((2, B), jnp.int32),
            jax.ShapeDtypeStruct((NPAD,), jnp.float32),
            jax.ShapeDtypeStruct((32,), jnp.int32),
        ],
        mesh=mesh,
        compiler_params=pltpu.CompilerParams(needs_layout_passes=False),
        scratch_types=[
            pltpu.VMEM((2, 128), jnp.int32),
            pltpu.VMEM((2, 128), jnp.float32),
            pltpu.VMEM((2, 128), jnp.int32),
            pltpu.VMEM((2, 128), jnp.int32),
            pltpu.VMEM((16,), jnp.int32),
            pltpu.VMEM((NS, 16), jnp.int32),
            pltpu.VMEM((SLT,), jnp.int32),
            pltpu.VMEM((SLT,), jnp.float32),
            pltpu.VMEM((GR,), jnp.int32),
            pltpu.VMEM((GC, HID), jnp.float32),
            pltpu.VMEM((GC, HID), jnp.float32),
            pltpu.VMEM((32,), jnp.int32),
            pltpu.VMEM_SHARED((NS, 16), jnp.int32),
            pltpu.VMEM_SHARED((NPAD,), jnp.int32),
            pltpu.VMEM_SHARED((NPAD,), jnp.float32),
            pltpu.SemaphoreType.DMA,
            pltpu.SemaphoreType.DMA,
            pltpu.SemaphoreType.DMA,
            pltpu.SemaphoreType.DMA,
        ],
    )(tke, tkw, x)

    ysel = pl.pallas_call(
        _expert_body,
        grid_spec=pltpu.PrefetchScalarGridSpec(
            num_scalar_prefetch=1,
            grid=(NB,),
            in_specs=[
                pl.BlockSpec((M, HID), lambda i, be: (i, 0)),
                pl.BlockSpec((1, HID, HID), lambda i, be: (be[i], 0, 0)),
                pl.BlockSpec((1, 1, HID), lambda i, be: (be[i], 0, 0)),
                pl.BlockSpec((M, 1), lambda i, be: (i, 0)),
            ],
            out_specs=pl.BlockSpec((M, HID), lambda i, be: (i, 0)),
        ),
        out_shape=jax.ShapeDtypeStruct((NPAD, HID), jnp.float32),
    )(bexp, xg, W_e, b_e.reshape(E, 1, HID), wvec.reshape(NPAD, 1))

    out = pl.kernel(
        _combine_body,
        out_type=jax.ShapeDtypeStruct((B, HID), jnp.float32),
        mesh=plsc.VectorSubcoreMesh(core_axis_name="c", subcore_axis_name="s",
                                    num_cores=NC, num_subcores=NS),
        scratch_types=[
            pltpu.VMEM((16,), jnp.int32),
            pltpu.VMEM((16,), jnp.int32),
            pltpu.VMEM((16, HID), jnp.float32),
            pltpu.VMEM((16, HID), jnp.float32),
            pltpu.VMEM((16, HID), jnp.float32),
            pltpu.VMEM((16, HID), jnp.float32),
            pltpu.VMEM((16, HID), jnp.float32),
            pltpu.VMEM((16, HID), jnp.float32),
            pltpu.SemaphoreType.DMA,
            pltpu.SemaphoreType.DMA,
        ],
    )(ysel, dests, base)
    return out, loss[0, 0]


# speedup vs baseline: 3.1695x; 3.0622x over previous
import jax
import jax.numpy as jnp
from jax.experimental import pallas as pl
from jax.experimental.pallas import tpu as pltpu

B = 2048
HID = 1024
E = 8
GD = 16
COEF = 0.01
S = E + 1


def _moe_body(x_ref, Wg1_ref, bg1_ref, Wg2_ref, bg2_ref, Wsh_ref, bsh_ref,
              shw_ref, rtw_ref, We_ref, be_ref, noise_ref,
              out_ref, loss_ref, m_ref, gsum_ref):
    s = pl.program_id(0)

    @pl.when(s == 0)
    def _gating():
        x = x_ref[...]
        h = jnp.maximum(
            jnp.dot(x, Wg1_ref[...].T, preferred_element_type=jnp.float32)
            + bg1_ref[...], 0.0)
        logits = (jnp.dot(h, Wg2_ref[...].T,
                          preferred_element_type=jnp.float32) + bg2_ref[...])
        gw = jax.nn.softmax(logits, axis=-1) + noise_ref[...]
        e_iota = jax.lax.broadcasted_iota(jnp.int32, (B, E), 1)
        i1 = jnp.argmax(gw, axis=1)
        m1 = jnp.max(gw, axis=1)
        masked = jnp.where(e_iota == i1[:, None], -jnp.inf, gw)
        i2 = jnp.argmax(masked, axis=1)
        m2 = jnp.max(masked, axis=1)
        denom = m1 + m2
        w1 = (m1 / denom)[:, None]
        w2 = (m2 / denom)[:, None]
        m_ref[...] = (jnp.where(e_iota == i1[:, None], w1, 0.0)
                      + jnp.where(e_iota == i2[:, None], w2, 0.0))
        gsum_ref[...] = jnp.sum(gw, axis=0, keepdims=True)

    @pl.when(s <= E - 1)
    def _expert():
        e = s
        x = x_ref[...]
        W = We_ref[0]
        z = jnp.dot(x, W.T, preferred_element_type=jnp.float32) + be_ref[0]
        y = jax.nn.sigmoid(z)
        sel = (jax.lax.broadcasted_iota(jnp.int32, (B, E), 1) == e)
        mcol = jnp.sum(jnp.where(sel, m_ref[...], 0.0), axis=1, keepdims=True)
        contrib = mcol * y

        @pl.when(s == 0)
        def _():
            out_ref[...] = contrib

        @pl.when(s > 0)
        def _():
            out_ref[...] += contrib

    @pl.when(s == S - 1)
    def _final():
        x = x_ref[...]
        sh = jax.nn.sigmoid(
            jnp.dot(x, Wsh_ref[...].T, preferred_element_type=jnp.float32)
            + bsh_ref[...])
        out_ref[...] = (x + shw_ref[0, 0] * sh
                        + rtw_ref[0, 0] * (out_ref[...] + x))
        p = gsum_ref[...] * (1.0 / B)
        loss_ref[...] = jnp.reshape(jnp.mean((1.0 / E - p) ** 2) * COEF,
                                    (1, 1))


def kernel(x, W_g1, b_g1, W_g2, b_g2, W_sh, b_sh, sh_w, rt_w, W_e, b_e, noise):
    const2 = lambda s: (0, 0)
    out, loss = pl.pallas_call(
        _moe_body,
        grid=(S,),
        in_specs=[
            pl.BlockSpec((B, HID), const2),
            pl.BlockSpec((GD, HID), const2),
            pl.BlockSpec((1, GD), const2),
            pl.BlockSpec((E, GD), const2),
            pl.BlockSpec((1, E), const2),
            pl.BlockSpec((HID, HID), const2),
            pl.BlockSpec((1, HID), const2),
            pl.BlockSpec((1, 1), const2),
            pl.BlockSpec((1, 1), const2),
            pl.BlockSpec((1, HID, HID),
                         lambda s: (jnp.clip(s, 0, E - 1), 0, 0)),
            pl.BlockSpec((1, 1, HID),
                         lambda s: (jnp.clip(s, 0, E - 1), 0, 0)),
            pl.BlockSpec((B, E), const2),
        ],
        out_specs=[
            pl.BlockSpec((B, HID), const2),
            pl.BlockSpec((1, 1), const2),
        ],
        out_shape=[
            jax.ShapeDtypeStruct((B, HID), jnp.float32),
            jax.ShapeDtypeStruct((1, 1), jnp.float32),
        ],
        scratch_shapes=[
            pltpu.VMEM((B, E), jnp.float32),
            pltpu.VMEM((1, E), jnp.float32),
        ],
    )(x, W_g1, b_g1.reshape(1, GD), W_g2, b_g2.reshape(1, E),
      W_sh, b_sh.reshape(1, HID), sh_w.reshape(1, 1), rt_w.reshape(1, 1),
      W_e, b_e.reshape(E, 1, HID), noise)
    return out, loss[0, 0]


# speedup vs baseline: 3.2205x; 1.0161x over previous
import jax
import jax.numpy as jnp
from jax.experimental import pallas as pl
from jax.experimental.pallas import tpu as pltpu

B = 2048
HID = 1024
E = 8
GD = 16
COEF = 0.01
S = E + 2


def _moe_body(x_ref, Wg1_ref, bg1_ref, Wg2_ref, bg2_ref, Wsh_ref, bsh_ref,
              shw_ref, rtw_ref, We_ref, be_ref, noise_ref,
              out_ref, loss_ref, m_ref, gsum_ref):
    s = pl.program_id(0)

    @pl.when(s == 0)
    def _gating():
        x = x_ref[...]
        h = jnp.maximum(
            jnp.dot(x, Wg1_ref[...].T, preferred_element_type=jnp.float32)
            + bg1_ref[...], 0.0)
        logits = (jnp.dot(h, Wg2_ref[...].T,
                          preferred_element_type=jnp.float32) + bg2_ref[...])
        gw = jax.nn.softmax(logits, axis=-1) + noise_ref[...]
        e_iota = jax.lax.broadcasted_iota(jnp.int32, (B, E), 1)
        i1 = jnp.argmax(gw, axis=1)
        m1 = jnp.max(gw, axis=1)
        masked = jnp.where(e_iota == i1[:, None], -jnp.inf, gw)
        i2 = jnp.argmax(masked, axis=1)
        m2 = jnp.max(masked, axis=1)
        denom = m1 + m2
        w1 = (m1 / denom)[:, None]
        w2 = (m2 / denom)[:, None]
        m_ref[...] = (jnp.where(e_iota == i1[:, None], w1, 0.0)
                      + jnp.where(e_iota == i2[:, None], w2, 0.0))
        gsum_ref[...] = jnp.sum(gw, axis=0, keepdims=True)

    @pl.when((s >= 1) & (s <= E))
    def _expert():
        e = s - 1
        x = x_ref[...]
        W = We_ref[0]
        z = jnp.dot(x, W.T, preferred_element_type=jnp.float32) + be_ref[0]
        y = jax.nn.sigmoid(z)
        sel = (jax.lax.broadcasted_iota(jnp.int32, (B, E), 1) == e)
        mcol = jnp.sum(jnp.where(sel, m_ref[...], 0.0), axis=1, keepdims=True)
        contrib = mcol * y

        @pl.when(s == 1)
        def _():
            out_ref[...] = contrib

        @pl.when(s > 1)
        def _():
            out_ref[...] += contrib

    @pl.when(s == S - 1)
    def _final():
        x = x_ref[...]
        sh = jax.nn.sigmoid(
            jnp.dot(x, Wsh_ref[...].T, preferred_element_type=jnp.float32)
            + bsh_ref[...])
        out_ref[...] = (x + shw_ref[0, 0] * sh
                        + rtw_ref[0, 0] * (out_ref[...] + x))
        p = gsum_ref[...] * (1.0 / B)
        loss_ref[...] = jnp.reshape(jnp.mean((1.0 / E - p) ** 2) * COEF,
                                    (1, 1))


def kernel(x, W_g1, b_g1, W_g2, b_g2, W_sh, b_sh, sh_w, rt_w, W_e, b_e, noise):
    const2 = lambda s: (0, 0)
    out, loss = pl.pallas_call(
        _moe_body,
        grid=(S,),
        in_specs=[
            pl.BlockSpec((B, HID), const2),
            pl.BlockSpec((GD, HID), const2),
            pl.BlockSpec((1, GD), const2),
            pl.BlockSpec((E, GD), const2),
            pl.BlockSpec((1, E), const2),
            pl.BlockSpec((HID, HID), const2),
            pl.BlockSpec((1, HID), const2),
            pl.BlockSpec((1, 1), const2),
            pl.BlockSpec((1, 1), const2),
            pl.BlockSpec((1, HID, HID),
                         lambda s: (jnp.clip(s - 1, 0, E - 1), 0, 0)),
            pl.BlockSpec((1, 1, HID),
                         lambda s: (jnp.clip(s - 1, 0, E - 1), 0, 0)),
            pl.BlockSpec((B, E), const2),
        ],
        out_specs=[
            pl.BlockSpec((B, HID), const2),
            pl.BlockSpec((1, 1), const2),
        ],
        out_shape=[
            jax.ShapeDtypeStruct((B, HID), jnp.float32),
            jax.ShapeDtypeStruct((1, 1), jnp.float32),
        ],
        scratch_shapes=[
            pltpu.VMEM((B, E), jnp.float32),
            pltpu.VMEM((1, E), jnp.float32),
        ],
    )(x, W_g1, b_g1.reshape(1, GD), W_g2, b_g2.reshape(1, E),
      W_sh, b_sh.reshape(1, HID), sh_w.reshape(1, 1), rt_w.reshape(1, 1),
      W_e, b_e.reshape(E, 1, HID), noise)
    return out, loss[0, 0]
